# baseline (speedup 1.0000x reference)
import functools

import jax
import jax.numpy as jnp
from jax import lax
from jax.experimental import pallas as pl
from jax.experimental.pallas import tpu as pltpu
from jax.experimental.pallas import tpu_sc as plsc

N = 100000
E = 1600000
G = 1024
F = 32
FH = 16
NCLS = 32

NSC = 2
NSUB = 16
NW = NSC * NSUB

BN = 2048
NBLK = 49
N_pad = BN * NBLK
CH = N_pad // NW
SUB = CH // 2

NDUMP = 128
H2 = N_pad + NDUMP
ZCH = H2 // NSUB
WCH = N_pad // NSUB

K = 256
EC = 100352
E_pad = EC * NSUB

_mesh = plsc.VectorSubcoreMesh(core_axis_name="core", subcore_axis_name="subcore")

_sc_params = pltpu.CompilerParams(
    needs_layout_passes=False, use_tc_tiling_on_sc=False)


def _embed_body(x0_hbm, x1_hbm, se_hbm, ce_hbm, h0_hbm, x0_v, x1_v, se_v, ce_v, hb_v):
    wid = lax.axis_index("subcore") * NSC + lax.axis_index("core")
    base = wid * CH
    pltpu.sync_copy(x0_hbm.at[pl.ds(base, CH)], x0_v)
    pltpu.sync_copy(x1_hbm.at[pl.ds(base, CH)], x1_v)
    pltpu.sync_copy(se_hbm, se_v)
    pltpu.sync_copy(ce_hbm, ce_v)
    iota = lax.iota(jnp.int32, 16)
    for plane, (tab_v, x_v) in enumerate(((se_v, x0_v), (ce_v, x1_v))):
        for half in range(2):
            @pl.loop(0, SUB, step=16)
            def _(v):
                row0 = half * SUB + v
                xv = x_v[pl.ds(row0, 16)]
                rows = v + iota
                for j in range(16):
                    cj = jnp.full((16,), j, jnp.int32)
                    col = plsc.load_gather(tab_v, [xv, cj])
                    plsc.store_scatter(hb_v, [rows, cj], col)
            pltpu.sync_copy(
                hb_v,
                h0_hbm.at[pl.ds(plane * N_pad + base + half * SUB, SUB)])


@jax.jit
def _embed(x0, x1, se, ce):
    kfn = pl.kernel(
        _embed_body,
        out_type=jax.ShapeDtypeStruct((2 * N_pad, FH), jnp.float32),
        mesh=_mesh,
        compiler_params=_sc_params,
        scratch_types=[
            pltpu.VMEM((CH,), jnp.int32),
            pltpu.VMEM((CH,), jnp.int32),
            pltpu.VMEM((128, 16), jnp.float32),
            pltpu.VMEM((128, 16), jnp.float32),
            pltpu.VMEM((SUB, FH), jnp.float32),
        ],
    )
    return kfn(x0, x1, se, ce)


def _agg_body(with_deg, *args):
    if with_deg:
        (h_hbm, s_hbm, d_hbm, z2_hbm, z1_hbm, agg_hbm, deg_hbm,
         sv0, dv0, iv0, rows0, sv1, dv1, iv1, rows1,
         lsem0, lsem1, gsem, ssem0, ssem1, ones_v, acc, accd) = args
    else:
        (h_hbm, s_hbm, d_hbm, z2_hbm, agg_hbm,
         sv0, dv0, iv0, rows0, sv1, dv1, iv1, rows1,
         lsem0, lsem1, gsem, ssem0, ssem1, acc) = args
    svs, dvs, ivs, rows_ = (sv0, sv1), (dv0, dv1), (iv0, iv1), (rows0, rows1)
    lsems, ssems = (lsem0, lsem1), (ssem0, ssem1)
    core = lax.axis_index("core")
    sub = lax.axis_index("subcore")
    pltpu.sync_copy(z2_hbm, acc.at[pl.ds(sub * ZCH, ZCH)])
    if with_deg:
        pltpu.sync_copy(z1_hbm, accd.at[pl.ds(sub * ZCH, ZCH)])

        @pl.loop(0, K, step=16)
        def _(q):
            ones_v[pl.ds(q, 16)] = jnp.full((16,), 1.0, jnp.float32)

    plsc.subcore_barrier()
    plane_base = core * N_pad
    tile_edge_base = sub * EC
    nw = EC // K

    def load(w, p):
        eb = tile_edge_base + w * K
        pltpu.async_copy(s_hbm.at[pl.ds(eb, K)], svs[p], lsems[p])
        pltpu.async_copy(d_hbm.at[pl.ds(eb, K)], dvs[p], lsems[p])

    def wait_load(p):
        pltpu.make_async_copy(s_hbm.at[pl.ds(0, K)], svs[p], lsems[p]).wait()
        pltpu.make_async_copy(d_hbm.at[pl.ds(0, K)], dvs[p], lsems[p]).wait()

    def wait_scatter(p):
        pltpu.make_async_copy(rows_[p], acc.at[ivs[p]], ssems[p]).wait()
        if with_deg:
            pltpu.make_async_copy(ones_v, accd.at[ivs[p]], ssems[p]).wait()

    load(0, 0)
    load(1, 1)

    @pl.loop(0, nw, step=2)
    def _(g):
        for p in range(2):
            w = g + p
            wait_load(p)

            @pl.loop(0, K, step=16)
            def _(q):
                d = dvs[p][pl.ds(q, 16)]
                valid = d >= 0
                ivs[p][pl.ds(q, 16)] = jnp.where(valid, d, N_pad + (d & (NDUMP - 1)))
                svs[p][pl.ds(q, 16)] = svs[p][pl.ds(q, 16)] + plane_base

            @pl.when(w >= 2)
            def _():
                wait_scatter(p)

            pltpu.async_copy(h_hbm.at[svs[p]], rows_[p], gsem).wait()
            pltpu.async_copy(rows_[p], acc.at[ivs[p]], ssems[p], add=True)
            if with_deg:
                pltpu.async_copy(ones_v, accd.at[ivs[p]], ssems[p], add=True)

            @pl.when(w + 2 < nw)
            def _():
                load(w + 2, p)

    wait_scatter(0)
    wait_scatter(1)
    plsc.subcore_barrier()
    pltpu.sync_copy(acc.at[pl.ds(sub * WCH, WCH)],
                    agg_hbm.at[pl.ds(plane_base + sub * WCH, WCH)])
    if with_deg:
        @pl.when(core == 0)
        def _():
            pltpu.sync_copy(accd.at[pl.ds(sub * WCH, WCH)],
                            deg_hbm.at[pl.ds(sub * WCH, WCH)])


@jax.jit
def _aggregate_deg(h, srcp, dstp, z2, z1):
    kfn = pl.kernel(
        functools.partial(_agg_body, True),
        out_type=(jax.ShapeDtypeStruct((2 * N_pad, FH), jnp.float32),
                  jax.ShapeDtypeStruct((N_pad,), jnp.float32)),
        mesh=_mesh,
        compiler_params=_sc_params,
        scratch_types=[
            pltpu.VMEM((K,), jnp.int32),
            pltpu.VMEM((K,), jnp.int32),
            pltpu.VMEM((K,), jnp.int32),
            pltpu.VMEM((K, FH), jnp.float32),
            pltpu.VMEM((K,), jnp.int32),
            pltpu.VMEM((K,), jnp.int32),
            pltpu.VMEM((K,), jnp.int32),
            pltpu.VMEM((K, FH), jnp.float32),
            pltpu.SemaphoreType.DMA,
            pltpu.SemaphoreType.DMA,
            pltpu.SemaphoreType.DMA,
            pltpu.SemaphoreType.DMA,
            pltpu.SemaphoreType.DMA,
            pltpu.VMEM((K,), jnp.float32),
            pltpu.VMEM_SHARED((H2, FH), jnp.float32),
            pltpu.VMEM_SHARED((H2,), jnp.float32),
        ],
    )
    return kfn(h, srcp, dstp, z2, z1)


@jax.jit
def _aggregate(h, srcp, dstp, z2):
    kfn = pl.kernel(
        functools.partial(_agg_body, False),
        out_type=jax.ShapeDtypeStruct((2 * N_pad, FH), jnp.float32),
        mesh=_mesh,
        compiler_params=_sc_params,
        scratch_types=[
            pltpu.VMEM((K,), jnp.int32),
            pltpu.VMEM((K,), jnp.int32),
            pltpu.VMEM((K,), jnp.int32),
            pltpu.VMEM((K, FH), jnp.float32),
            pltpu.VMEM((K,), jnp.int32),
            pltpu.VMEM((K,), jnp.int32),
            pltpu.VMEM((K,), jnp.int32),
            pltpu.VMEM((K, FH), jnp.float32),
            pltpu.SemaphoreType.DMA,
            pltpu.SemaphoreType.DMA,
            pltpu.SemaphoreType.DMA,
            pltpu.SemaphoreType.DMA,
            pltpu.SemaphoreType.DMA,
            pltpu.VMEM_SHARED((H2, FH), jnp.float32),
        ],
    )
    return kfn(h, srcp, dstp, z2)


def _dense_body(agg_ref, deg_ref, h_ref, wl_ref, b_ref, wr_ref, out_ref):
    agg = jnp.concatenate([agg_ref[0], agg_ref[1]], axis=-1)
    h = jnp.concatenate([h_ref[0], h_ref[1]], axis=-1)
    mean = agg / jnp.maximum(deg_ref[...], 1.0)[:, None]
    out = (lax.dot_general(mean, wl_ref[...], (((1,), (1,)), ((), ())),
                           preferred_element_type=jnp.float32)
           + lax.dot_general(h, wr_ref[...], (((1,), (1,)), ((), ())),
                             preferred_element_type=jnp.float32)
           + b_ref[...])
    out = jnp.maximum(out, 0.0)
    out_ref[0] = out[:, :FH]
    out_ref[1] = out[:, FH:]


@jax.jit
def _dense(agg, deg, h, wl, b, wr):
    return pl.pallas_call(
        _dense_body,
        grid=(NBLK,),
        in_specs=[
            pl.BlockSpec((2, BN, FH), lambda i: (0, i, 0)),
            pl.BlockSpec((BN,), lambda i: (i,)),
            pl.BlockSpec((2, BN, FH), lambda i: (0, i, 0)),
            pl.BlockSpec((F, F), lambda i: (0, 0)),
            pl.BlockSpec((1, F), lambda i: (0, 0)),
            pl.BlockSpec((F, F), lambda i: (0, 0)),
        ],
        out_specs=pl.BlockSpec((2, BN, FH), lambda i: (0, i, 0)),
        out_shape=jax.ShapeDtypeStruct((2, N_pad, FH), jnp.float32),
    )(agg, deg, h, wl, b, wr)


def _pool_body(h_ref, batch_ref, wc_ref, bc_ref, out_ref, acc_s, acc_c):
    i = pl.program_id(0)

    @pl.when(i == 0)
    def _():
        acc_s[...] = jnp.zeros_like(acc_s)
        acc_c[...] = jnp.zeros_like(acc_c)

    h = jnp.concatenate([h_ref[0], h_ref[1]], axis=-1)
    ids = batch_ref[...]
    gi = lax.broadcasted_iota(jnp.int32, (G, BN), 0)
    oh = (gi == ids[None, :]).astype(jnp.float32)
    acc_s[...] += lax.dot_general(oh, h, (((1,), (0,)), ((), ())),
                                  preferred_element_type=jnp.float32)
    acc_c[...] += jnp.sum(oh, axis=1)

    @pl.when(i == NBLK - 1)
    def _():
        hg = acc_s[...] / jnp.maximum(acc_c[...], 1.0)[:, None]
        out_ref[...] = lax.dot_general(hg, wc_ref[...], (((1,), (1,)), ((), ())),
                                       preferred_element_type=jnp.float32) + bc_ref[...]


@jax.jit
def _pool(h, batch, wc, bc):
    return pl.pallas_call(
        _pool_body,
        grid=(NBLK,),
        in_specs=[
            pl.BlockSpec((2, BN, FH), lambda i: (0, i, 0)),
            pl.BlockSpec((BN,), lambda i: (i,)),
            pl.BlockSpec((NCLS, F), lambda i: (0, 0)),
            pl.BlockSpec((1, NCLS), lambda i: (0, 0)),
        ],
        out_specs=pl.BlockSpec((G, NCLS), lambda i: (0, 0)),
        out_shape=jax.ShapeDtypeStruct((G, NCLS), jnp.float32),
        scratch_shapes=[
            pltpu.VMEM((G, F), jnp.float32),
            pltpu.VMEM((G,), jnp.float32),
        ],
    )(h, batch, wc, bc)


def kernel(x, edge_index, batch, shape_emb, color_emb, W1l, b1, W1r, W2l, b2, W2r, Wc, bc):
    x = x.astype(jnp.int32)
    x0 = jnp.pad(x[:, 0], (0, N_pad - N))
    x1 = jnp.pad(x[:, 1], (0, N_pad - N))
    src = jnp.pad(edge_index[0], (0, E_pad - E))
    dst = jnp.pad(edge_index[1], (0, E_pad - E), constant_values=-1)
    batch_p = jnp.pad(batch, (0, N_pad - N), constant_values=G)
    z2 = jnp.zeros((ZCH, FH), jnp.float32)
    z1 = jnp.zeros((ZCH,), jnp.float32)

    h0 = _embed(x0, x1, shape_emb, color_emb)
    a1, deg = _aggregate_deg(h0, src, dst, z2, z1)
    h1 = _dense(a1.reshape(2, N_pad, FH), deg, h0.reshape(2, N_pad, FH),
                W1l, b1.reshape(1, F), W1r)
    a2 = _aggregate(h1.reshape(2 * N_pad, FH), src, dst, z2)
    h2 = _dense(a2.reshape(2, N_pad, FH), deg, h1, W2l, b2.reshape(1, F), W2r)
    return _pool(h2, batch_p, Wc, bc.reshape(1, NCLS))


# speedup vs baseline: 1.1282x; 1.1282x over previous
import dataclasses
import functools

import jax
import jax.numpy as jnp
from jax import lax
from jax.experimental import pallas as pl
from jax.experimental.pallas import tpu as pltpu
from jax.experimental.pallas import tpu_sc as plsc

N = 100000
E = 1600000
G = 1024
F = 32
NCLS = 32

NSC = 2
NSUB = 16
NW = NSC * NSUB

BN = 2048
NBLK = 49
N_pad = BN * NBLK
CH = N_pad // NW
SUB = CH // 2

H = N_pad // NSC
NDUMP = 128
H2 = H + NDUMP
ZCH = H2 // NSUB

K = 256
EC = 100352
E_pad = EC * NSUB

_mesh = plsc.VectorSubcoreMesh(core_axis_name="core", subcore_axis_name="subcore")

_sc_params = pltpu.CompilerParams(
    needs_layout_passes=False, use_tc_tiling_on_sc=False)


def _embed_body(x0_hbm, x1_hbm, se_hbm, ce_hbm, h0_hbm, x0_v, x1_v, se_v, ce_v, hb_v):
    wid = lax.axis_index("subcore") * NSC + lax.axis_index("core")
    base = wid * CH
    pltpu.sync_copy(x0_hbm.at[pl.ds(base, CH)], x0_v)
    pltpu.sync_copy(x1_hbm.at[pl.ds(base, CH)], x1_v)
    pltpu.sync_copy(se_hbm, se_v)
    pltpu.sync_copy(ce_hbm, ce_v)
    iota = lax.iota(jnp.int32, 16)
    for half in range(2):
        @pl.loop(0, SUB, step=16)
        def _(v):
            row0 = half * SUB + v
            xv0 = x0_v[pl.ds(row0, 16)]
            xv1 = x1_v[pl.ds(row0, 16)]
            rows = v + iota
            for j in range(16):
                cj = jnp.full((16,), j, jnp.int32)
                s_col = plsc.load_gather(se_v, [xv0, cj])
                plsc.store_scatter(hb_v, [rows, cj], s_col)
                c_col = plsc.load_gather(ce_v, [xv1, cj])
                plsc.store_scatter(hb_v, [rows, cj + 16], c_col)
        pltpu.sync_copy(hb_v, h0_hbm.at[pl.ds(base + half * SUB, SUB)])


@jax.jit
def _embed(x0, x1, se, ce):
    kfn = pl.kernel(
        _embed_body,
        out_type=jax.ShapeDtypeStruct((N_pad, F), jnp.float32),
        mesh=_mesh,
        compiler_params=_sc_params,
        scratch_types=[
            pltpu.VMEM((CH,), jnp.int32),
            pltpu.VMEM((CH,), jnp.int32),
            pltpu.VMEM((128, 16), jnp.float32),
            pltpu.VMEM((128, 16), jnp.float32),
            pltpu.VMEM((SUB, F), jnp.float32),
        ],
    )
    return kfn(x0, x1, se, ce)


def _agg_body(with_deg, *args):
    if with_deg:
        (h_hbm, s_hbm, d_hbm, z2_hbm, z1_hbm, agg_hbm, deg_hbm,
         sv0, dv0, sv1, dv1, cs_f, cd_f, st_s, st_d, rows,
         lsem0, lsem1, gsem, ssem, ones_v, acc, accd) = args
    else:
        (h_hbm, s_hbm, d_hbm, z2_hbm, agg_hbm,
         sv0, dv0, sv1, dv1, cs_f, cd_f, st_s, st_d, rows,
         lsem0, lsem1, gsem, ssem, acc) = args
    svs, dvs = (sv0, sv1), (dv0, dv1)
    lsems = (lsem0, lsem1)
    core = lax.axis_index("core")
    sub = lax.axis_index("subcore")
    pltpu.sync_copy(z2_hbm, acc.at[pl.ds(sub * ZCH, ZCH)])
    if with_deg:
        pltpu.sync_copy(z1_hbm, accd.at[pl.ds(sub * ZCH, ZCH)])

        @pl.loop(0, K, step=16)
        def _(q):
            ones_v[pl.ds(q, 16)] = jnp.full((16,), 1.0, jnp.float32)

    plsc.subcore_barrier()
    half_base = core * H
    tile_edge_base = sub * EC
    nw = EC // K
    iota16 = lax.iota(jnp.int32, 16)

    def load(w, p):
        eb = tile_edge_base + w * K
        pltpu.async_copy(s_hbm.at[pl.ds(eb, K)], svs[p], lsems[p])
        pltpu.async_copy(d_hbm.at[pl.ds(eb, K)], dvs[p], lsems[p])

    def wait_load(p):
        pltpu.make_async_copy(s_hbm.at[pl.ds(0, K)], svs[p], lsems[p]).wait()
        pltpu.make_async_copy(d_hbm.at[pl.ds(0, K)], dvs[p], lsems[p]).wait()

    def wait_scatter():
        pltpu.make_async_copy(rows, acc.at[st_d], ssem).wait()
        if with_deg:
            pltpu.make_async_copy(ones_v, accd.at[st_d], ssem).wait()

    def flush(nf):
        @pl.when(nf > 0)
        def _():
            wait_scatter()

        @pl.loop(0, K, step=16)
        def _(q):
            st_s[pl.ds(q, 16)] = cs_f[pl.ds(q, 16)]
            st_d[pl.ds(q, 16)] = cd_f[pl.ds(q, 16)]

        pltpu.async_copy(h_hbm.at[st_s], rows, gsem).wait()
        pltpu.async_copy(rows, acc.at[st_d], ssem, add=True)
        if with_deg:
            pltpu.async_copy(ones_v, accd.at[st_d], ssem, add=True)
        ts = cs_f[pl.ds(K, 16)]
        td = cd_f[pl.ds(K, 16)]
        cs_f[pl.ds(0, 16)] = ts
        cd_f[pl.ds(0, 16)] = td

    load(0, 0)
    load(1, 1)

    def outer(i, carry):
        off, nf = carry
        for p in range(2):
            w = 2 * i + p
            wait_load(p)
            for q in range(0, K, 16):
                s = svs[p][pl.ds(q, 16)]
                dl = dvs[p][pl.ds(q, 16)] - half_base
                mask = (dl >= 0) & (dl < H)
                plsc.store_compressed(cs_f.at[pl.ds(off, 16)], s, mask=mask)
                plsc.store_compressed(cd_f.at[pl.ds(off, 16)], dl, mask=mask)
                cnt = jnp.max(plsc.all_reduce_population_count(mask))
                off = off + cnt

                def _do_flush(c):
                    o, n = c
                    flush(n)
                    return o - K, n + 1

                off, nf = lax.cond(off >= K, _do_flush, lambda c: c, (off, nf))

            @pl.when(w + 2 < nw)
            def _():
                load(w + 2, p)
        return off, nf

    off, nf = lax.fori_loop(0, nw // 2, outer, (jnp.int32(0), jnp.int32(0)))

    @pl.loop(0, K, step=16)
    def _(q):
        lane = q + iota16
        sel = lane >= off
        cs_f[pl.ds(q, 16)] = jnp.where(sel, 0, cs_f[pl.ds(q, 16)])
        cd_f[pl.ds(q, 16)] = jnp.where(sel, H + (lane & (NDUMP - 1)),
                                       cd_f[pl.ds(q, 16)])

    flush(nf)
    wait_scatter()
    plsc.subcore_barrier()
    out_base = core * H + sub * (H // NSUB)
    pltpu.sync_copy(acc.at[pl.ds(sub * (H // NSUB), H // NSUB)],
                    agg_hbm.at[pl.ds(out_base, H // NSUB)])
    if with_deg:
        pltpu.sync_copy(accd.at[pl.ds(sub * (H // NSUB), H // NSUB)],
                        deg_hbm.at[pl.ds(out_base, H // NSUB)])


@jax.jit
def _aggregate_deg(h, srcp, dstp, z2, z1):
    kfn = pl.kernel(
        functools.partial(_agg_body, True),
        out_type=(jax.ShapeDtypeStruct((N_pad, F), jnp.float32),
                  jax.ShapeDtypeStruct((N_pad,), jnp.float32)),
        mesh=_mesh,
        compiler_params=_sc_params,
        scratch_types=[
            pltpu.VMEM((K,), jnp.int32),
            pltpu.VMEM((K,), jnp.int32),
            pltpu.VMEM((K,), jnp.int32),
            pltpu.VMEM((K,), jnp.int32),
            pltpu.VMEM((K + 16,), jnp.int32),
            pltpu.VMEM((K + 16,), jnp.int32),
            pltpu.VMEM((K,), jnp.int32),
            pltpu.VMEM((K,), jnp.int32),
            pltpu.VMEM((K, F), jnp.float32),
            pltpu.SemaphoreType.DMA,
            pltpu.SemaphoreType.DMA,
            pltpu.SemaphoreType.DMA,
            pltpu.SemaphoreType.DMA,
            pltpu.VMEM((K,), jnp.float32),
            pltpu.VMEM_SHARED((H2, F), jnp.float32),
            pltpu.VMEM_SHARED((H2,), jnp.float32),
        ],
    )
    return kfn(h, srcp, dstp, z2, z1)


@jax.jit
def _aggregate(h, srcp, dstp, z2):
    kfn = pl.kernel(
        functools.partial(_agg_body, False),
        out_type=jax.ShapeDtypeStruct((N_pad, F), jnp.float32),
        mesh=_mesh,
        compiler_params=_sc_params,
        scratch_types=[
            pltpu.VMEM((K,), jnp.int32),
            pltpu.VMEM((K,), jnp.int32),
            pltpu.VMEM((K,), jnp.int32),
            pltpu.VMEM((K,), jnp.int32),
            pltpu.VMEM((K + 16,), jnp.int32),
            pltpu.VMEM((K + 16,), jnp.int32),
            pltpu.VMEM((K,), jnp.int32),
            pltpu.VMEM((K,), jnp.int32),
            pltpu.VMEM((K, F), jnp.float32),
            pltpu.SemaphoreType.DMA,
            pltpu.SemaphoreType.DMA,
            pltpu.SemaphoreType.DMA,
            pltpu.SemaphoreType.DMA,
            pltpu.VMEM_SHARED((H2, F), jnp.float32),
        ],
    )
    return kfn(h, srcp, dstp, z2)


def _dense_body(agg_ref, deg_ref, h_ref, wl_ref, b_ref, wr_ref, out_ref):
    mean = agg_ref[...] / jnp.maximum(deg_ref[...], 1.0)[:, None]
    out = (lax.dot_general(mean, wl_ref[...], (((1,), (1,)), ((), ())),
                           preferred_element_type=jnp.float32)
           + lax.dot_general(h_ref[...], wr_ref[...], (((1,), (1,)), ((), ())),
                             preferred_element_type=jnp.float32)
           + b_ref[...])
    out_ref[...] = jnp.maximum(out, 0.0)


@jax.jit
def _dense(agg, deg, h, wl, b, wr):
    return pl.pallas_call(
        _dense_body,
        grid=(NBLK,),
        in_specs=[
            pl.BlockSpec((BN, F), lambda i: (i, 0)),
            pl.BlockSpec((BN,), lambda i: (i,)),
            pl.BlockSpec((BN, F), lambda i: (i, 0)),
            pl.BlockSpec((F, F), lambda i: (0, 0)),
            pl.BlockSpec((1, F), lambda i: (0, 0)),
            pl.BlockSpec((F, F), lambda i: (0, 0)),
        ],
        out_specs=pl.BlockSpec((BN, F), lambda i: (i, 0)),
        out_shape=jax.ShapeDtypeStruct((N_pad, F), jnp.float32),
    )(agg, deg, h, wl, b, wr)


def _pool_body(h_ref, batch_ref, wc_ref, bc_ref, out_ref, acc_s, acc_c):
    i = pl.program_id(0)

    @pl.when(i == 0)
    def _():
        acc_s[...] = jnp.zeros_like(acc_s)
        acc_c[...] = jnp.zeros_like(acc_c)

    ids = batch_ref[...]
    gi = lax.broadcasted_iota(jnp.int32, (G, BN), 0)
    oh = (gi == ids[None, :]).astype(jnp.float32)
    acc_s[...] += lax.dot_general(oh, h_ref[...], (((1,), (0,)), ((), ())),
                                  preferred_element_type=jnp.float32)
    acc_c[...] += jnp.sum(oh, axis=1)

    @pl.when(i == NBLK - 1)
    def _():
        hg = acc_s[...] / jnp.maximum(acc_c[...], 1.0)[:, None]
        out_ref[...] = lax.dot_general(hg, wc_ref[...], (((1,), (1,)), ((), ())),
                                       preferred_element_type=jnp.float32) + bc_ref[...]


@jax.jit
def _pool(h, batch, wc, bc):
    return pl.pallas_call(
        _pool_body,
        grid=(NBLK,),
        in_specs=[
            pl.BlockSpec((BN, F), lambda i: (i, 0)),
            pl.BlockSpec((BN,), lambda i: (i,)),
            pl.BlockSpec((NCLS, F), lambda i: (0, 0)),
            pl.BlockSpec((1, NCLS), lambda i: (0, 0)),
        ],
        out_specs=pl.BlockSpec((G, NCLS), lambda i: (0, 0)),
        out_shape=jax.ShapeDtypeStruct((G, NCLS), jnp.float32),
        scratch_shapes=[
            pltpu.VMEM((G, F), jnp.float32),
            pltpu.VMEM((G,), jnp.float32),
        ],
    )(h, batch, wc, bc)


def kernel(x, edge_index, batch, shape_emb, color_emb, W1l, b1, W1r, W2l, b2, W2r, Wc, bc):
    x = x.astype(jnp.int32)
    x0 = jnp.pad(x[:, 0], (0, N_pad - N))
    x1 = jnp.pad(x[:, 1], (0, N_pad - N))
    src = jnp.pad(edge_index[0], (0, E_pad - E))
    dst = jnp.pad(edge_index[1], (0, E_pad - E), constant_values=-1)
    batch_p = jnp.pad(batch, (0, N_pad - N), constant_values=G)
    z2 = jnp.zeros((ZCH, F), jnp.float32)
    z1 = jnp.zeros((ZCH,), jnp.float32)

    h0 = _embed(x0, x1, shape_emb, color_emb)
    a1, deg = _aggregate_deg(h0, src, dst, z2, z1)
    h1 = _dense(a1, deg, h0, W1l, b1.reshape(1, F), W1r)
    a2 = _aggregate(h1, src, dst, z2)
    h2 = _dense(a2, deg, h1, W2l, b2.reshape(1, F), W2r)
    return _pool(h2, batch_p, Wc, bc.reshape(1, NCLS))
